# Initial kernel scaffold; baseline (speedup 1.0000x reference)
import jax
import jax.numpy as jnp
from jax.experimental import pallas as pl


def kernel(pan, ms, Wp, Ws, Wg, Wu, Wd):
    raise NotImplementedError("write your pallas kernel here")


# speedup vs baseline: 3.1514x; 3.1514x over previous
import jax
import jax.numpy as jnp
from jax.experimental import pallas as pl

PATCH = 32
NUM_EXPERTS = 16
DIM = 1024
ALPHA = 0.001
NEG_INF = float("-inf")
HIGHEST = jax.lax.Precision.HIGHEST


def _pool_kernel(pan_ref, ms_ref, out_ref):
    n = 256 // PATCH
    q = jax.lax.broadcasted_iota(jnp.int32, (n * PATCH, PATCH), 0)
    j = jax.lax.broadcasted_iota(jnp.int32, (n * PATCH, PATCH), 1)
    fold = (q % PATCH == j).astype(jnp.float32)
    scale = 1.0 / (n * n)
    for c in range(5):
        xr = pan_ref[0, 0] if c == 0 else ms_ref[0, c - 1]
        xb = xr.astype(jnp.bfloat16).astype(jnp.float32)
        s1 = xb[0:PATCH, :]
        for gh in range(1, n):
            s1 = s1 + xb[gh * PATCH:(gh + 1) * PATCH, :]
        s2 = jnp.dot(s1, fold, preferred_element_type=jnp.float32,
                     precision=HIGHEST)
        out_ref[0, c] = s2 * scale


def _router_kernel(pooled_ref, wp_ref, ws_ref, h_ref, idx_ref, aux_ref):
    pooled = pooled_ref[...]
    wpb = wp_ref[...].astype(jnp.bfloat16).astype(jnp.float32)
    h = jnp.dot(pooled, wpb, preferred_element_type=jnp.float32,
                precision=HIGHEST)
    h_ref[...] = h
    logits = jnp.dot(h.astype(jnp.bfloat16), ws_ref[...].astype(jnp.bfloat16),
                     preferred_element_type=jnp.float32)

    B, E = logits.shape
    k = idx_ref.shape[1]
    iota = jax.lax.broadcasted_iota(jnp.int32, (B, E), 1)
    vals = logits
    counts = jnp.zeros((B, E), jnp.float32)
    for slot in range(k):
        mx = jnp.max(vals, axis=1, keepdims=True)
        at_max = vals >= mx
        idx = jnp.min(jnp.where(at_max, iota, E), axis=1, keepdims=True)
        sel = iota == idx
        idx_ref[:, slot:slot + 1] = idx
        counts = counts + sel.astype(jnp.float32)
        vals = jnp.where(sel, NEG_INF, vals)

    m = jnp.max(logits, axis=1, keepdims=True)
    ex = jnp.exp(logits - m)
    p = ex / jnp.sum(ex, axis=1, keepdims=True)

    ce = jnp.sum(counts, axis=0, keepdims=True) / (B * k)
    pi = jnp.sum(p, axis=0, keepdims=True) / B
    aux_ref[...] = jnp.reshape(jnp.sum(pi * ce) * (E * ALPHA), (1, 1))


def _expert_kernel(h_ref, idx_ref, wg_ref, wu_ref, wd_ref, out_ref):
    e = pl.program_id(0)
    hb = h_ref[...].astype(jnp.bfloat16)
    g = jnp.dot(hb, wg_ref[0].astype(jnp.bfloat16),
                preferred_element_type=jnp.float32)
    u = jnp.dot(hb, wu_ref[0].astype(jnp.bfloat16),
                preferred_element_type=jnp.float32)
    a = (g * jax.nn.sigmoid(g)) * u
    dn = jnp.dot(a.astype(jnp.bfloat16), wd_ref[0].astype(jnp.bfloat16),
                 preferred_element_type=jnp.float32)
    oe = jnp.maximum(dn, 0.0)
    mask = (idx_ref[...] == e).astype(jnp.float32)
    contrib = oe[:, None, :] * mask[:, :, None]

    @pl.when(e == 0)
    def _init():
        out_ref[...] = contrib

    @pl.when(e > 0)
    def _acc():
        out_ref[...] += contrib


@jax.jit
def kernel(pan, ms, Wp, Ws, Wg, Wu, Wd):
    B = pan.shape[0]
    k = ms.shape[1]
    C = 1 + ms.shape[1]
    E = Ws.shape[1]

    pooled = pl.pallas_call(
        _pool_kernel,
        grid=(B,),
        in_specs=[
            pl.BlockSpec((1, 1, 256, 256), lambda b: (b, 0, 0, 0)),
            pl.BlockSpec((1, 4, 256, 256), lambda b: (b, 0, 0, 0)),
        ],
        out_specs=pl.BlockSpec((1, C, PATCH, PATCH), lambda b: (b, 0, 0, 0)),
        out_shape=jax.ShapeDtypeStruct((B, C, PATCH, PATCH), jnp.float32),
    )(pan, ms)
    pooled = pooled.reshape(B, C * PATCH * PATCH)

    h, topk_idx, aux = pl.pallas_call(
        _router_kernel,
        in_specs=[
            pl.BlockSpec(pooled.shape, lambda: (0, 0)),
            pl.BlockSpec(Wp.shape, lambda: (0, 0)),
            pl.BlockSpec(Ws.shape, lambda: (0, 0)),
        ],
        out_specs=[
            pl.BlockSpec((B, DIM), lambda: (0, 0)),
            pl.BlockSpec((B, k), lambda: (0, 0)),
            pl.BlockSpec((1, 1), lambda: (0, 0)),
        ],
        out_shape=[
            jax.ShapeDtypeStruct((B, DIM), jnp.float32),
            jax.ShapeDtypeStruct((B, k), jnp.int32),
            jax.ShapeDtypeStruct((1, 1), jnp.float32),
        ],
    )(pooled, Wp, Ws)

    selected = pl.pallas_call(
        _expert_kernel,
        grid=(E,),
        in_specs=[
            pl.BlockSpec((B, DIM), lambda e: (0, 0)),
            pl.BlockSpec((B, k), lambda e: (0, 0)),
            pl.BlockSpec((1, DIM, DIM), lambda e: (e, 0, 0)),
            pl.BlockSpec((1, DIM, DIM), lambda e: (e, 0, 0)),
            pl.BlockSpec((1, DIM, DIM), lambda e: (e, 0, 0)),
        ],
        out_specs=pl.BlockSpec((B, k, DIM), lambda e: (0, 0, 0)),
        out_shape=jax.ShapeDtypeStruct((B, k, DIM), jnp.float32),
    )(h, topk_idx, Wg, Wu, Wd)

    return selected, aux[0, 0]
